# baseline (speedup 1.0000x reference)
import functools

import jax
import jax.numpy as jnp
from jax import lax
from jax.experimental import pallas as pl
from jax.experimental.pallas import tpu as pltpu
from jax.experimental.pallas import tpu_sc as plsc

N = 10000
E = 320000
H = 128
L = 5
OUT = 2
BN_EPS = 1e-5

NC = 2
NS = 16
HH = H // NC
RANGE = 10080
EPAD = 32 * RANGE
CHUNK = 80
NCHUNK = 2 * RANGE // CHUNK
RING = 2
NPAD = 10240
ZROWS = 128
ROWS_PER_SUB = NPAD // NS
EMB_PER_W = NPAD // (NC * NS)
EMB_CHUNKS = EMB_PER_W // CHUNK
INV_N = 9.999999747378752e-05


def _sc_agg_body(src_hbm, dst_hbm, h2_hbm, out_hbm, src_v, dst_v, rows_v,
                 zbuf, stage, idx16, agg_sh, s0, s1):
    c = lax.axis_index("c")
    s = lax.axis_index("s")
    sems = [s0, s1]

    def _zrow(i, carry):
        for j in range(HH // 16):
            zbuf[i, pl.ds(j * 16, 16)] = jnp.zeros((16,), jnp.float32)
        return carry
    lax.fori_loop(0, ZROWS, _zrow, 0)
    for t in range(ROWS_PER_SUB // ZROWS):
        pltpu.sync_copy(
            zbuf, agg_sh.at[pl.ds(s * ROWS_PER_SUB + t * ZROWS, ZROWS)])
    plsc.subcore_barrier()

    pltpu.sync_copy(src_hbm.at[s], src_v)
    pltpu.sync_copy(dst_hbm.at[s], dst_v)

    def _sxform(i, carry):
        for j in range(CHUNK // 16):
            v = src_v[i, pl.ds(j * 16, 16)]
            src_v[i, pl.ds(j * 16, 16)] = v * 2 + c
        return carry
    lax.fori_loop(0, NCHUNK, _sxform, 0)

    for r in range(RING):
        pltpu.async_copy(h2_hbm.at[src_v.at[r]],
                         rows_v.at[pl.ds(r * CHUNK, CHUNK)], sems[r])

    zero16 = jnp.zeros((16,), jnp.float32)
    nvec = HH // 16
    lanes = lax.iota(jnp.int32, 16)
    garbage = jnp.full((16,), NPAD - 1, jnp.int32)

    def _advance(carry, do_push):
        prev, slot = carry
        t = jnp.where(do_push, slot, jnp.int32(16))
        idx16[...] = jnp.where(lanes == t, prev, idx16[...])
        newslot = jnp.where(do_push, slot + 1, slot)

        @pl.when(newslot == 16)
        def _():
            pltpu.sync_copy(stage, agg_sh.at[idx16], add=True)

        slot = jnp.where(newslot == 16, jnp.int32(0), newslot)
        return slot

    def _edge(k, r, e_carry):
        carry = e_carry
        rbase = lax.rem(k, jnp.int32(RING)) * CHUNK
        for vi in range(CHUNK // 16):
            dvec = dst_v[k, pl.ds(vi * 16, 16)]
            for lane in range(16):
                e = vi * 16 + lane
                prev = carry[0]
                d = dvec[lane]
                is_new = d != prev
                slot = _advance(carry, jnp.logical_and(is_new, prev >= 0))
                mv = lanes >= jnp.where(is_new, jnp.int32(0), jnp.int32(16))
                for j in range(nvec):
                    row = rows_v[rbase + e, pl.ds(j * 16, 16)]
                    old = stage[slot, pl.ds(j * 16, 16)]
                    stage[slot, pl.ds(j * 16, 16)] = jnp.where(
                        mv, row, old + row)
                carry = (d, slot)
        return carry

    def _group(gi, carry):
        for r in range(RING):
            k = gi * RING + r
            pltpu.make_async_copy(h2_hbm.at[src_v.at[k]],
                                  rows_v.at[pl.ds(r * CHUNK, CHUNK)],
                                  sems[r]).wait()

            do_f = jnp.logical_and(k == NCHUNK // 2, carry[0] >= 0)
            slot = _advance(carry, do_f)
            prev = jnp.where(do_f, jnp.int32(-1), carry[0])
            carry = (prev, slot)

            carry = _edge(k, r, carry)
            kn = k + RING

            @pl.when(kn < NCHUNK)
            def _():
                pltpu.async_copy(h2_hbm.at[src_v.at[kn]],
                                 rows_v.at[pl.ds(r * CHUNK, CHUNK)], sems[r])
        return carry

    idx16[...] = garbage
    init = (jnp.int32(-1), jnp.int32(0))
    carry = lax.fori_loop(0, NCHUNK // RING, _group, init)

    slot = _advance(carry, carry[0] >= 0)

    @pl.when(slot > 0)
    def _():
        idx16[...] = jnp.where(lanes < slot, idx16[...], garbage)
        pltpu.sync_copy(stage, agg_sh.at[idx16], add=True)

    plsc.subcore_barrier()
    pltpu.sync_copy(agg_sh.at[pl.ds(s * ROWS_PER_SUB, ROWS_PER_SUB)],
                    out_hbm.at[pl.ds(s * ROWS_PER_SUB, ROWS_PER_SUB), c])


def _sc_embed_body(state_hbm, emb_hbm, out_hbm, idx_v, rows_v, sem):
    c = lax.axis_index("c")
    s = lax.axis_index("s")
    wid = s * NC + c
    pltpu.sync_copy(state_hbm.at[wid], idx_v)
    for k in range(EMB_CHUNKS):
        pltpu.async_copy(emb_hbm.at[idx_v.at[k]], rows_v, sem).wait()
        pltpu.sync_copy(
            rows_v, out_hbm.at[pl.ds(wid * EMB_PER_W + k * CHUNK, CHUNK)])


@functools.cache
def _sc_kernels():
    mesh = plsc.VectorSubcoreMesh(core_axis_name="c", subcore_axis_name="s",
                                  num_cores=NC, num_subcores=NS)
    agg = pl.kernel(
        _sc_agg_body,
        out_type=jax.ShapeDtypeStruct((NPAD, NC, HH), jnp.float32),
        mesh=mesh,
        compiler_params=pltpu.CompilerParams(use_tc_tiling_on_sc=False),
        scratch_types=[
            pltpu.VMEM((NCHUNK, CHUNK), jnp.int32),
            pltpu.VMEM((NCHUNK, CHUNK), jnp.int32),
            pltpu.VMEM((RING * CHUNK, HH), jnp.float32),
            pltpu.VMEM((ZROWS, HH), jnp.float32),
            pltpu.VMEM((16, HH), jnp.float32),
            pltpu.VMEM((16,), jnp.int32),
            pltpu.VMEM_SHARED((NPAD, HH), jnp.float32),
            pltpu.SemaphoreType.DMA,
            pltpu.SemaphoreType.DMA,
        ],
    )
    embed = pl.kernel(
        _sc_embed_body,
        out_type=jax.ShapeDtypeStruct((NPAD, H), jnp.float32),
        mesh=mesh,
        scratch_types=[
            pltpu.VMEM((EMB_CHUNKS, CHUNK), jnp.int32),
            pltpu.VMEM((CHUNK, H), jnp.float32),
            pltpu.SemaphoreType.DMA,
        ],
    )
    return agg, embed


def _col_mean_ref(xref, f):
    def win(b):
        def body(i, acc):
            return acc + f(xref[pl.ds(b + 8 * i, 8), :])
        acc = lax.fori_loop(1, 625, body, f(xref[pl.ds(b, 8), :]))
        a4 = acc[0:4] + acc[4:8]
        a2 = a4[0:2] + a4[2:4]
        return a2[0:1] + a2[1:2]
    return (win(0) + win(5000)) * INV_N


def _bn_ref(xref, g, b):
    mu = _col_mean_ref(xref, lambda sl: sl)
    var = _col_mean_ref(xref, lambda sl: (sl - mu) * (sl - mu))
    return (xref[...] - mu) / jnp.sqrt(var + BN_EPS) * g + b


def _mlp_body(h_ref, a_ref, w1_ref, w2_ref, p_ref, out_ref, u_ref):
    z = h_ref[...] + a_ref[...]
    u_ref[...] = jnp.dot(z, w1_ref[...], preferred_element_type=jnp.float32)
    u = jnp.maximum(_bn_ref(u_ref, p_ref[0:1, :], p_ref[1:2, :]), 0.0)
    u_ref[...] = jnp.dot(u, w2_ref[...], preferred_element_type=jnp.float32)
    out_ref[...] = jnp.maximum(
        _bn_ref(u_ref, p_ref[2:3, :], p_ref[3:4, :]), 0.0)


_tc_mlp = pl.pallas_call(
    _mlp_body,
    out_shape=jax.ShapeDtypeStruct((N, H), jnp.float32),
    scratch_shapes=[pltpu.VMEM((N, H), jnp.float32)],
)


def _readout_body(cat_ref, wr1_ref, wr2_ref, b_ref, out_ref):
    s = jnp.dot(cat_ref[...], wr1_ref[...],
                preferred_element_type=jnp.float32) + b_ref[0:1, :]
    y = jnp.maximum(s, 0.0)
    out_ref[...] = (
        jnp.dot(y, wr2_ref[...], preferred_element_type=jnp.float32)
        + b_ref[1:2, :])


_RB = 2000
_tc_readout = pl.pallas_call(
    _readout_body,
    grid=(N // _RB,),
    in_specs=[
        pl.BlockSpec((_RB, L * H), lambda i: (i, 0)),
        pl.BlockSpec((L * H, H), lambda i: (0, 0)),
        pl.BlockSpec((H, H), lambda i: (0, 0)),
        pl.BlockSpec((8, H), lambda i: (0, 0)),
    ],
    out_specs=pl.BlockSpec((_RB, H), lambda i: (i, 0)),
    out_shape=jax.ShapeDtypeStruct((N, H), jnp.float32),
)


def kernel(state, edge_index, emb, W1, W2, g1, b1, g2, b2, Wr1, br1, Wr2, br2):
    src = edge_index[0].astype(jnp.int32)
    dst = edge_index[1].astype(jnp.int32)
    perm = jnp.argsort(dst, stable=True)
    ssrc = jnp.concatenate(
        [src[perm], jnp.zeros((EPAD - E,), jnp.int32)]).reshape(
            NS, NCHUNK, CHUNK)
    sdst = jnp.concatenate(
        [dst[perm], jnp.full((EPAD - E,), NPAD - 1, jnp.int32)]).reshape(
            NS, NCHUNK, CHUNK)

    state_p = jnp.pad(state.astype(jnp.int32), (0, NPAD - N))
    state_p = state_p.reshape(NC * NS, EMB_CHUNKS, CHUNK)

    sc_agg, sc_embed = _sc_kernels()
    h = sc_embed(state_p, emb)[:N]
    hs = [h]
    for i in range(L - 1):
        h2 = h.reshape(2 * N, HH)
        agg = sc_agg(ssrc, sdst, h2).reshape(NPAD, H)[:N]
        p = jnp.stack([g1[i], b1[i], g2[i], b2[i]], axis=0)
        p = jnp.concatenate([p, jnp.zeros((4, H), jnp.float32)], axis=0)
        h = _tc_mlp(h, agg, W1[i], W2[i], p)
        hs.append(h)

    cat = jnp.concatenate(hs, axis=-1)
    wr2 = jnp.zeros((H, H), jnp.float32).at[:, :OUT].set(Wr2)
    b = jnp.zeros((8, H), jnp.float32)
    b = b.at[0, :].set(br1)
    b = b.at[1, :OUT].set(br2)
    score = _tc_readout(cat, Wr1, wr2, b)
    return score[:, :OUT]
